# Initial kernel scaffold; baseline (speedup 1.0000x reference)
import jax
import jax.numpy as jnp
from jax.experimental import pallas as pl


def kernel(relative_position_bias_table, relative_position_index):
    raise NotImplementedError("write your pallas kernel here")


# speedup vs baseline: 2.0168x; 2.0168x over previous
import functools

import jax
import jax.numpy as jnp
from jax import lax
from jax.experimental import pallas as pl
from jax.experimental.pallas import tpu as pltpu
from jax.experimental.pallas import tpu_sc as plsc

L = 1025
NH = 16
NREL = 3972
N = L * L
NW = 32
PERW = 32768
BLK = 2048
NBLK = PERW // BLK
EXTRA = NW * PERW
TAIL = EXTRA + 16 * 128


_mesh = plsc.VectorSubcoreMesh(core_axis_name="c", subcore_axis_name="s")


@functools.partial(
    pl.kernel,
    mesh=_mesh,
    out_type=(
        jax.ShapeDtypeStruct((NH, N), jnp.float32),
        jax.ShapeDtypeStruct((NH, 128), jnp.float32),
    ),
    scratch_types=[
        pltpu.VMEM((NREL * NH,), jnp.float32),
        pltpu.VMEM((BLK,), jnp.int32),
        pltpu.VMEM((NH, BLK), jnp.float32),
    ],
    compiler_params=pltpu.CompilerParams(needs_layout_passes=False),
)
def _gather_bias(tab_hbm, idx_hbm, tail_hbm, out_hbm, tailout_hbm,
                 tab_v, idx_v, out_v):
    wid = lax.axis_index("s") * 2 + lax.axis_index("c")

    pltpu.sync_copy(tab_hbm.at[pl.ds(0, NREL * NH)], tab_v)

    base_w = wid * PERW

    def block(b, carry):
        base = base_w + b * BLK
        pltpu.sync_copy(idx_hbm.at[pl.ds(base, BLK)], idx_v)

        def chunk(c, carry2):
            off = c * 16
            iv = idx_v[pl.ds(off, 16)] * 16
            for h in range(NH):
                out_v[h, pl.ds(off, 16)] = plsc.load_gather(tab_v, [iv + h])
            return carry2

        lax.fori_loop(0, BLK // 16, chunk, 0)
        pltpu.sync_copy(out_v, out_hbm.at[:, pl.ds(base, BLK)])
        return carry

    lax.fori_loop(0, NBLK, block, 0)

    @pl.when(wid < 16)
    def _extra():
        base = EXTRA + wid * 128
        pltpu.sync_copy(idx_hbm.at[pl.ds(base, 128)], idx_v.at[pl.ds(0, 128)])

        def chunk(c, carry2):
            off = c * 16
            iv = idx_v[pl.ds(off, 16)] * 16
            for h in range(NH):
                out_v[h, pl.ds(off, 16)] = plsc.load_gather(tab_v, [iv + h])
            return carry2

        lax.fori_loop(0, 8, chunk, 0)
        pltpu.sync_copy(out_v.at[:, pl.ds(0, 128)],
                        out_hbm.at[:, pl.ds(base, 128)])

    @pl.when(wid == 0)
    def _tail():
        pltpu.sync_copy(tail_hbm.at[pl.ds(0, 16)], idx_v.at[pl.ds(0, 16)])
        iv = idx_v[pl.ds(0, 16)] * 16
        for h in range(NH):
            out_v[h, pl.ds(0, 16)] = plsc.load_gather(tab_v, [iv + h])
        pltpu.sync_copy(out_v.at[:, pl.ds(0, 128)],
                        tailout_hbm.at[:, pl.ds(0, 128)])


@jax.jit
def kernel(relative_position_bias_table, relative_position_index):
    tab = relative_position_bias_table.reshape(-1)
    idx = relative_position_index.reshape(-1)
    tail = jnp.pad(idx[N - 1:], (0, 15))
    out, tl = _gather_bias(tab, idx, tail)
    out = lax.dynamic_update_slice(out, tl[:, :1], (0, TAIL))
    return out.reshape(1, NH, L, L)


# speedup vs baseline: 2.1376x; 1.0599x over previous
import functools

import jax
import jax.numpy as jnp
from jax import lax
from jax.experimental import pallas as pl
from jax.experimental.pallas import tpu as pltpu
from jax.experimental.pallas import tpu_sc as plsc

L = 1025
NH = 16
NREL = 3972
N = L * L
NW = 32
PERW = 32768
BLK = 2048
NBLK = PERW // BLK
EXTRA = NW * PERW
TAIL = EXTRA + 16 * 128


_mesh = plsc.VectorSubcoreMesh(core_axis_name="c", subcore_axis_name="s")


@functools.partial(
    pl.kernel,
    mesh=_mesh,
    out_type=(
        jax.ShapeDtypeStruct((NH, N), jnp.float32),
        jax.ShapeDtypeStruct((NH, 128), jnp.float32),
    ),
    scratch_types=[
        pltpu.VMEM((NREL * NH,), jnp.float32),
        pltpu.VMEM((BLK,), jnp.int32),
        pltpu.VMEM((NH, BLK), jnp.float32),
    ],
    compiler_params=pltpu.CompilerParams(needs_layout_passes=False),
)
def _gather_bias(tab_hbm, idx_hbm, tail_hbm, out_hbm, tailout_hbm,
                 tab_v, idx_v, out_v):
    wid = lax.axis_index("s") * 2 + lax.axis_index("c")

    pltpu.sync_copy(tab_hbm.at[pl.ds(0, NREL * NH)], tab_v)

    base_w = wid * PERW

    def block(b, carry):
        base = base_w + b * BLK
        pltpu.sync_copy(idx_hbm.at[pl.ds(base, BLK)], idx_v)

        def chunk(c, carry2):
            off = c * 16
            iv = idx_v[pl.ds(off, 16)]
            for h in range(NH):
                out_v[h, pl.ds(off, 16)] = plsc.load_gather(
                    tab_v, [iv + h * NREL])
            return carry2

        lax.fori_loop(0, BLK // 16, chunk, 0)
        pltpu.sync_copy(out_v, out_hbm.at[:, pl.ds(base, BLK)])
        return carry

    lax.fori_loop(0, NBLK, block, 0)

    @pl.when(wid < 16)
    def _extra():
        base = EXTRA + wid * 128
        pltpu.sync_copy(idx_hbm.at[pl.ds(base, 128)], idx_v.at[pl.ds(0, 128)])

        def chunk(c, carry2):
            off = c * 16
            iv = idx_v[pl.ds(off, 16)]
            for h in range(NH):
                out_v[h, pl.ds(off, 16)] = plsc.load_gather(
                    tab_v, [iv + h * NREL])
            return carry2

        lax.fori_loop(0, 8, chunk, 0)
        pltpu.sync_copy(out_v.at[:, pl.ds(0, 128)],
                        out_hbm.at[:, pl.ds(base, 128)])

    @pl.when(wid == 0)
    def _tail():
        pltpu.sync_copy(tail_hbm.at[pl.ds(0, 16)], idx_v.at[pl.ds(0, 16)])
        iv = idx_v[pl.ds(0, 16)]
        for h in range(NH):
            out_v[h, pl.ds(0, 16)] = plsc.load_gather(tab_v, [iv + h * NREL])
        pltpu.sync_copy(out_v.at[:, pl.ds(0, 128)],
                        tailout_hbm.at[:, pl.ds(0, 128)])


@jax.jit
def kernel(relative_position_bias_table, relative_position_index):
    tab = relative_position_bias_table.T.reshape(-1)
    idx = relative_position_index.reshape(-1)
    tail = jnp.pad(idx[N - 1:], (0, 15))
    out, tl = _gather_bias(tab, idx, tail)
    out = lax.dynamic_update_slice(out, tl[:, :1], (0, TAIL))
    return out.reshape(1, NH, L, L)


# speedup vs baseline: 13.3608x; 6.2503x over previous
import functools

import jax
import jax.numpy as jnp
from jax import lax
from jax.experimental import pallas as pl
from jax.experimental.pallas import tpu as pltpu
from jax.experimental.pallas import tpu_sc as plsc

L = 1025
NH = 16
NREL = 3972
NW = 32
GPW = 4
NCH = (L - 1) // 16
CTAIL = L - 16


_mesh = plsc.VectorSubcoreMesh(core_axis_name="c", subcore_axis_name="s")


@functools.partial(
    pl.kernel,
    mesh=_mesh,
    out_type=(
        jax.ShapeDtypeStruct((NH, L, L), jnp.float32),
        jax.ShapeDtypeStruct((NH, L), jnp.float32),
    ),
    scratch_types=[
        pltpu.VMEM((NREL * NH,), jnp.float32),
        pltpu.VMEM((8, L), jnp.int32),
        pltpu.VMEM((4, 8, L), jnp.float32),
        pltpu.VMEM((NH, L), jnp.float32),
    ],
    compiler_params=pltpu.CompilerParams(needs_layout_passes=False),
)
def _gather_bias(tab_hbm, idx_hbm, out_hbm, out2_hbm,
                 tab_v, idx_v, out_v, out2_v):
    wid = lax.axis_index("s") * 2 + lax.axis_index("c")

    pltpu.sync_copy(tab_hbm.at[pl.ds(0, NREL * NH)], tab_v)

    cidx = lax.iota(jnp.int32, 16) + CTAIL

    def group(g, carry):
        r0 = (wid * GPW + g) * 8
        pltpu.sync_copy(idx_hbm.at[pl.ds(r0, 8), :], idx_v)
        for hg in range(4):
            for rr in range(8):
                def chunk(c, carry2):
                    off = c * 16
                    iv = idx_v[rr, pl.ds(off, 16)]
                    for k in range(4):
                        out_v[k, rr, pl.ds(off, 16)] = plsc.load_gather(
                            tab_v, [iv + (hg * 4 + k) * NREL])
                    return carry2

                lax.fori_loop(0, NCH, chunk, 0, unroll=4)
                rsp = jnp.full((16,), rr, jnp.int32)
                iv = plsc.load_gather(idx_v, [rsp, cidx])
                for k in range(4):
                    vals = plsc.load_gather(tab_v,
                                            [iv + (hg * 4 + k) * NREL])
                    plsc.store_scatter(
                        out_v, [jnp.full((16,), k, jnp.int32), rsp, cidx],
                        vals)
            pltpu.sync_copy(out_v,
                            out_hbm.at[pl.ds(hg * 4, 4), pl.ds(r0, 8), :])
        return carry

    lax.fori_loop(0, GPW, group, 0)

    @pl.when(wid == 0)
    def _stray():
        pltpu.sync_copy(idx_hbm.at[pl.ds(1024, 8), :], idx_v)
        rsp0 = jnp.full((16,), 0, jnp.int32)
        iv_t = plsc.load_gather(idx_v, [rsp0, cidx])
        for h in range(NH):
            def chunk(c, carry2):
                off = c * 16
                iv = idx_v[0, pl.ds(off, 16)]
                out2_v[h, pl.ds(off, 16)] = plsc.load_gather(
                    tab_v, [iv + h * NREL])
                return carry2

            lax.fori_loop(0, NCH, chunk, 0, unroll=4)
            vals = plsc.load_gather(tab_v, [iv_t + h * NREL])
            plsc.store_scatter(out2_v, [jnp.full((16,), h, jnp.int32), cidx],
                               vals)
        pltpu.sync_copy(out2_v, out2_hbm.at[pl.ds(0, NH), :])


@jax.jit
def kernel(relative_position_bias_table, relative_position_index):
    tab = relative_position_bias_table.T.reshape(-1)
    idx = jnp.pad(relative_position_index, ((0, 7), (0, 0)))
    out, row_last = _gather_bias(tab, idx)
    out = lax.dynamic_update_slice(
        out, row_last.reshape(NH, 1, L), (0, L - 1, 0))
    return out.reshape(1, NH, L, L)
